# baseline (speedup 1.0000x reference)
import jax
import jax.numpy as jnp
from jax import lax
from jax.experimental import pallas as pl
from jax.experimental.pallas import tpu as pltpu
from jax.experimental.pallas import tpu_sc as plsc

_NC = 2
_NS = 16
_NW = _NC * _NS
_CH_ROWS = 16
_D = 1024


def _sc_body(in_hbm, emb_hbm, out_hbm,
             vin0, vin1, vemb0, vemb1, vout0, vout1,
             sin0, sin1, semb0, semb1, sout0, sout1):
    batch, seq_len, _ = in_hbm.shape
    rows_per_w = seq_len // (_NW // batch)
    n_chunks = rows_per_w // _CH_ROWS
    w_per_batch = _NW // batch

    wid = lax.axis_index("s") * _NC + lax.axis_index("c")
    bb = wid // w_per_batch
    s_base = (wid % w_per_batch) * rows_per_w

    vin = (vin0, vin1)
    vemb = (vemb0, vemb1)
    vout = (vout0, vout1)
    sin = (sin0, sin1)
    semb = (semb0, semb1)
    sout = (sout0, sout1)

    def start_in(c):
        b = c & 1
        s0 = s_base + c * _CH_ROWS
        d_in = pltpu.async_copy(
            in_hbm.at[bb, pl.ds(s0, _CH_ROWS), :], vin[b], sin[b])
        d_emb = pltpu.async_copy(
            emb_hbm.at[pl.ds(s0, _CH_ROWS), :], vemb[b], semb[b])
        return d_in, d_emb

    in_descs = {0: start_in(0), 1: start_in(1)}
    out_descs = {}

    for c in range(n_chunks):
        b = c & 1
        d_in, d_emb = in_descs.pop(c)
        d_in.wait()
        d_emb.wait()
        if c >= 2:
            out_descs.pop(c - 2).wait()

        @plsc.parallel_loop(0, _CH_ROWS * _D, step=16, unroll=8)
        def _(i):
            r = i >> 10
            col = pl.multiple_of(i & (_D - 1), 16)
            vout[b][r, pl.ds(col, 16)] = (
                vin[b][r, pl.ds(col, 16)] + vemb[b][r, pl.ds(col, 16)])

        s0 = s_base + c * _CH_ROWS
        out_descs[c] = pltpu.async_copy(
            vout[b], out_hbm.at[bb, pl.ds(s0, _CH_ROWS), :], sout[b])
        if c + 2 < n_chunks:
            in_descs[c + 2] = start_in(c + 2)

    out_descs.pop(n_chunks - 2).wait()
    out_descs.pop(n_chunks - 1).wait()


def kernel(inputs, embeddings):
    batch, seq_len, dim = inputs.shape
    pos = embeddings[:seq_len]
    mesh = plsc.VectorSubcoreMesh(
        core_axis_name="c", subcore_axis_name="s",
        num_cores=_NC, num_subcores=_NS)
    run = pl.kernel(
        _sc_body,
        out_type=jax.ShapeDtypeStruct((batch, seq_len, dim), jnp.float32),
        mesh=mesh,
        compiler_params=pltpu.CompilerParams(
            use_tc_tiling_on_sc=True, skip_device_barrier=True),
        scratch_types=[
            pltpu.VMEM((_CH_ROWS, dim), jnp.float32),
            pltpu.VMEM((_CH_ROWS, dim), jnp.float32),
            pltpu.VMEM((_CH_ROWS, dim), jnp.float32),
            pltpu.VMEM((_CH_ROWS, dim), jnp.float32),
            pltpu.VMEM((_CH_ROWS, dim), jnp.float32),
            pltpu.VMEM((_CH_ROWS, dim), jnp.float32),
            pltpu.SemaphoreType.DMA,
            pltpu.SemaphoreType.DMA,
            pltpu.SemaphoreType.DMA,
            pltpu.SemaphoreType.DMA,
            pltpu.SemaphoreType.DMA,
            pltpu.SemaphoreType.DMA,
        ],
    )
    return run(inputs, pos)


# speedup vs baseline: 2.3910x; 2.3910x over previous
import jax
import jax.numpy as jnp
from jax.experimental import pallas as pl


_S_BLK = 512


def _add_kernel(in_ref, emb_ref, out_ref):
    out_ref[...] = in_ref[...] + emb_ref[...][None, :, :]


def kernel(inputs, embeddings):
    batch, seq_len, dim = inputs.shape
    pos = embeddings[:seq_len]
    grid = (seq_len // _S_BLK,)
    return pl.pallas_call(
        _add_kernel,
        grid=grid,
        in_specs=[
            pl.BlockSpec((batch, _S_BLK, dim), lambda i: (0, i, 0)),
            pl.BlockSpec((_S_BLK, dim), lambda i: (i, 0)),
        ],
        out_specs=pl.BlockSpec((batch, _S_BLK, dim), lambda i: (0, i, 0)),
        out_shape=jax.ShapeDtypeStruct((batch, seq_len, dim), inputs.dtype),
    )(inputs, pos)
